# Initial kernel scaffold; baseline (speedup 1.0000x reference)
import jax
import jax.numpy as jnp
from jax.experimental import pallas as pl


def kernel(x, edge_index, W1, b1, W2, b2):
    raise NotImplementedError("write your pallas kernel here")


# speedup vs baseline: 9.9397x; 9.9397x over previous
import functools

import jax
import jax.numpy as jnp
from jax import lax
from jax.experimental import pallas as pl
from jax.experimental.pallas import tpu as pltpu
from jax.experimental.pallas import tpu_sc as plsc

N = 10000
E = 320000
D = 128
NC = 2
NS = 16
NW = NC * NS
CHUNK = 64
NCHUNK = (E + NW * CHUNK - 1) // (NW * CHUNK)
NCHUNK += NCHUNK % 2
EPAD = NW * NCHUNK * CHUNK
RPT = 640
NPAD = RPT * NS

_MESH = plsc.VectorSubcoreMesh(core_axis_name="c", subcore_axis_name="s")


@functools.partial(
    pl.kernel,
    out_type=jax.ShapeDtypeStruct((NC, NPAD, D), jnp.float32),
    mesh=_MESH,
    scratch_types=[
        pltpu.VMEM_SHARED((NPAD, D), jnp.float32),
        pltpu.VMEM((CHUNK,), jnp.int32),
        pltpu.VMEM((CHUNK,), jnp.int32),
        pltpu.VMEM((CHUNK, D), jnp.float32),
        pltpu.VMEM((CHUNK, D), jnp.float32),
        pltpu.SemaphoreType.DMA,
        pltpu.SemaphoreType.DMA,
    ],
)
def _sc_degree(dst_hbm, ones_hbm, zeros_hbm, deg_out, deg_acc, di0, di1,
               ones_v, stage_v, s0, s1):
    cid = lax.axis_index("c")
    sid = lax.axis_index("s")
    wid = sid * NC + cid
    pltpu.sync_copy(zeros_hbm, stage_v)
    pltpu.sync_copy(ones_hbm, ones_v)

    def zbody(k, carry):
        pltpu.sync_copy(stage_v, deg_acc.at[pl.ds(sid * RPT + k * CHUNK, CHUNK)])
        return carry

    lax.fori_loop(0, RPT // CHUNK, zbody, 0)
    plsc.subcore_barrier()

    def body(i, carry):
        j0 = 2 * i
        pltpu.sync_copy(dst_hbm.at[wid, j0], di0)
        pltpu.sync_copy(dst_hbm.at[wid, j0 + 1], di1)
        cs0 = pltpu.async_copy(ones_v, deg_acc.at[di0], s0, add=True)
        cs1 = pltpu.async_copy(ones_v, deg_acc.at[di1], s1, add=True)
        cs0.wait()
        cs1.wait()
        return carry

    lax.fori_loop(0, NCHUNK // 2, body, 0)
    plsc.subcore_barrier()

    def obody(k, carry):
        r = pl.ds(sid * RPT + k * CHUNK, CHUNK)
        pltpu.sync_copy(deg_acc.at[r], stage_v)
        pltpu.sync_copy(stage_v, deg_out.at[cid, r])
        return carry

    lax.fori_loop(0, RPT // CHUNK, obody, 0)


@functools.partial(
    pl.kernel,
    out_type=jax.ShapeDtypeStruct((NC, NPAD, D), jnp.float32),
    mesh=_MESH,
    scratch_types=[
        pltpu.VMEM_SHARED((NPAD, D), jnp.float32),
        pltpu.VMEM((CHUNK,), jnp.int32),
        pltpu.VMEM((CHUNK,), jnp.int32),
        pltpu.VMEM((CHUNK,), jnp.int32),
        pltpu.VMEM((CHUNK,), jnp.int32),
        pltpu.VMEM((CHUNK, D), jnp.float32),
        pltpu.VMEM((CHUNK, D), jnp.float32),
        pltpu.SemaphoreType.DMA,
        pltpu.SemaphoreType.DMA,
        pltpu.SemaphoreType.DMA,
        pltpu.SemaphoreType.DMA,
    ],
)
def _sc_propagate(src_hbm, dst_hbm, xs_hbm, zeros_hbm, out_hbm,
                  acc, si0, si1, di0, di1, r0, r1, g0, g1, s0, s1):
    cid = lax.axis_index("c")
    sid = lax.axis_index("s")
    wid = sid * NC + cid
    pltpu.sync_copy(zeros_hbm, r0)

    def zbody(k, carry):
        pltpu.sync_copy(r0, acc.at[pl.ds(sid * RPT + k * CHUNK, CHUNK)])
        return carry

    lax.fori_loop(0, RPT // CHUNK, zbody, 0)
    plsc.subcore_barrier()

    def body(i, carry):
        j0 = 2 * i
        j1 = j0 + 1
        pltpu.sync_copy(src_hbm.at[wid, j0], si0)
        pltpu.sync_copy(src_hbm.at[wid, j1], si1)
        cg0 = pltpu.async_copy(xs_hbm.at[si0], r0, g0)
        cg1 = pltpu.async_copy(xs_hbm.at[si1], r1, g1)
        pltpu.sync_copy(dst_hbm.at[wid, j0], di0)
        pltpu.sync_copy(dst_hbm.at[wid, j1], di1)
        cg0.wait()
        cs0 = pltpu.async_copy(r0, acc.at[di0], s0, add=True)
        cg1.wait()
        cs1 = pltpu.async_copy(r1, acc.at[di1], s1, add=True)
        cs0.wait()
        cs1.wait()
        return carry

    lax.fori_loop(0, NCHUNK // 2, body, 0)
    plsc.subcore_barrier()

    def obody(k, carry):
        r = pl.ds(sid * RPT + k * CHUNK, CHUNK)
        pltpu.sync_copy(acc.at[r], r0)
        pltpu.sync_copy(r0, out_hbm.at[cid, r])
        return carry

    lax.fori_loop(0, RPT // CHUNK, obody, 0)


_TR = 1000
_TG = N // _TR

_DOT = dict(preferred_element_type=jnp.float32, precision=lax.Precision.HIGHEST)


def _tc_scale_mm1(dega_ref, degb_ref, x_ref, w1_ref, dinv_ref, xs_ref):
    deg = dega_ref[:, :1] + degb_ref[:, :1] + 1.0
    dinv = lax.rsqrt(deg)
    dinv_ref[...] = jnp.broadcast_to(dinv, (_TR, D))
    xs_ref[...] = jnp.dot(x_ref[...], w1_ref[...], **_DOT) * dinv


def _tc_relu_mm2(p1a_ref, p1b_ref, xs1_ref, dinv_ref, w2_ref, b1_ref, xs2_ref):
    dinv = dinv_ref[...]
    s = dinv * (p1a_ref[...] + p1b_ref[...] + xs1_ref[...]) + b1_ref[...]
    h = jnp.maximum(s, 0.0)
    xs2_ref[...] = jnp.dot(h, w2_ref[...], **_DOT) * dinv


def _tc_final(p2a_ref, p2b_ref, xs2_ref, dinv_ref, b2_ref, z_ref):
    z_ref[...] = (dinv_ref[...] * (p2a_ref[...] + p2b_ref[...] + xs2_ref[...])
                  + b2_ref[...])


def _row_spec(width):
    return pl.BlockSpec((_TR, width), lambda i: (i, 0))


def _full_spec(shape):
    return pl.BlockSpec(shape, lambda i: (0,) * len(shape))


_tc1 = pl.pallas_call(
    _tc_scale_mm1,
    grid=(_TG,),
    in_specs=[_row_spec(D), _row_spec(D), _row_spec(D), _full_spec((D, D))],
    out_specs=[_row_spec(D), _row_spec(D)],
    out_shape=[jax.ShapeDtypeStruct((N, D), jnp.float32)] * 2,
)

_tc2 = pl.pallas_call(
    _tc_relu_mm2,
    grid=(_TG,),
    in_specs=[_row_spec(D), _row_spec(D), _row_spec(D), _row_spec(D),
              _full_spec((D, D)), _full_spec((1, D))],
    out_specs=_row_spec(D),
    out_shape=jax.ShapeDtypeStruct((N, D), jnp.float32),
)

_tc3 = pl.pallas_call(
    _tc_final,
    grid=(_TG,),
    in_specs=[_row_spec(D), _row_spec(D), _row_spec(D), _row_spec(D),
              _full_spec((1, D))],
    out_specs=_row_spec(D),
    out_shape=jax.ShapeDtypeStruct((N, D), jnp.float32),
)


def kernel(x, edge_index, W1, b1, W2, b2):
    src = edge_index[0].astype(jnp.int32)
    dst = edge_index[1].astype(jnp.int32)
    npad = EPAD - E
    src_p = jnp.concatenate([src, jnp.zeros((npad,), jnp.int32)])
    dst_p = jnp.concatenate([dst, jnp.full((npad,), N, jnp.int32)])
    src_w = src_p.reshape(NW, NCHUNK, CHUNK)
    dst_w = dst_p.reshape(NW, NCHUNK, CHUNK)

    onesd = jnp.ones((CHUNK, D), jnp.float32)
    zerosd = jnp.zeros((CHUNK, D), jnp.float32)

    degp = _sc_degree(dst_w, onesd, zerosd)
    dinv128, xs1 = _tc1(degp[0, :N], degp[1, :N], x, W1)
    p1 = _sc_propagate(src_w, dst_w, xs1, zerosd)
    xs2 = _tc2(p1[0, :N], p1[1, :N], xs1, dinv128, W2, b1.reshape(1, D))
    p2 = _sc_propagate(src_w, dst_w, xs2, zerosd)
    return _tc3(p2[0, :N], p2[1, :N], xs2, dinv128, b2.reshape(1, D))
